# baseline (speedup 1.0000x reference)
import functools

import jax
import jax.numpy as jnp
from jax import lax
from jax.experimental import pallas as pl
from jax.experimental.pallas import tpu as pltpu
from jax.experimental.pallas import tpu_sc as plsc

B = 16384
F = 26
NNUM = 13
D = 32
FIELD_SIZE = 100000
NROWS = F + NNUM
NC, NS, L = 2, 16, 16
NW = NC * NS
BPW = B // NW
NB = 16
NCHUNK = BPW // NB
GCH = NB * F
NCH = NB * NNUM
VG = 2600000


def _body(xc_hbm, xn_hbm, w_hbm, bias_hbm, table_hbm, pat_hbm, pcat_hbm,
          pnum_hbm, out_hbm,
          idx_v, pat_v, pcat_v, pnum_v, dcat_v, dnum_v, grp_v, cat_v,
          num_v, xn_v, w_v, b_v, gsem, ssem):
  wid = lax.axis_index("s") * NC + lax.axis_index("c")
  base = wid * BPW

  pltpu.sync_copy(pat_hbm, pat_v)
  pltpu.sync_copy(pcat_hbm, pcat_v)
  pltpu.sync_copy(pnum_hbm, pnum_v)
  pltpu.sync_copy(w_hbm, w_v)
  pltpu.sync_copy(bias_hbm, b_v)

  def chunk(c, carry):
    b0 = base + c * NB
    pltpu.sync_copy(xc_hbm.at[pl.ds(b0 * F, GCH)], idx_v)

    def prep(g, _):
      s = pl.ds(g * L, L)
      idx_v[s] = idx_v[s] + pat_v[s]
      dcat_v[s] = pcat_v[s] + jnp.full((L,), b0 * NROWS, jnp.int32)
      return 0
    lax.fori_loop(0, GCH // L, prep, 0)

    gather = pltpu.async_copy(table_hbm.at[idx_v], grp_v, gsem)

    off = jnp.full((L,), b0 * NROWS, jnp.int32)

    def dnum(g, _):
      s = pl.ds(g * L, L)
      dnum_v[s] = pnum_v[s] + off
      return 0
    lax.fori_loop(0, NCH // L, dnum, 0)

    pltpu.sync_copy(xn_hbm.at[pl.ds(b0 * 16, NB * 16)], xn_v)
    for j in range(NNUM):
      w0 = w_v[j, pl.ds(0, L)]
      w1 = w_v[j, pl.ds(L, L)]
      bb0 = b_v[j, pl.ds(0, L)]
      bb1 = b_v[j, pl.ds(L, L)]

      def nbody(r, _, w0=w0, w1=w1, bb0=bb0, bb1=bb1, j=j):
        xv = xn_v[pl.ds(r * 16, L)]
        xb = xv.at[jnp.full((L,), j, jnp.int32)].get(mode="promise_in_bounds")
        rj = r * NNUM + j
        num_v[rj, pl.ds(0, L)] = xb * w0 + bb0
        num_v[rj, pl.ds(L, L)] = xb * w1 + bb1
        return 0
      lax.fori_loop(0, NB, nbody, 0)

    gather.wait()

    def extract(r, _):
      cat_v[r, pl.ds(0, L)] = grp_v[r, pl.ds(0, L)]
      cat_v[r, pl.ds(L, L)] = grp_v[r, pl.ds(L, L)]
      return 0
    lax.fori_loop(0, GCH, extract, 0)

    sc = pltpu.async_copy(cat_v, out_hbm.at[dcat_v], ssem)
    sn = pltpu.async_copy(num_v, out_hbm.at[dnum_v], ssem)
    sc.wait()
    sn.wait()
    return 0

  lax.fori_loop(0, NCHUNK, chunk, 0)


_embedder = functools.partial(
    pl.kernel,
    out_type=jax.ShapeDtypeStruct((B * NROWS, D), jnp.float32),
    mesh=plsc.VectorSubcoreMesh(
        core_axis_name="c", subcore_axis_name="s",
        num_cores=NC, num_subcores=NS),
    compiler_params=pltpu.CompilerParams(use_tc_tiling_on_sc=False),
    scratch_types=[
        pltpu.VMEM((GCH,), jnp.int32),
        pltpu.VMEM((GCH,), jnp.int32),
        pltpu.VMEM((GCH,), jnp.int32),
        pltpu.VMEM((NCH,), jnp.int32),
        pltpu.VMEM((GCH,), jnp.int32),
        pltpu.VMEM((NCH,), jnp.int32),
        pltpu.VMEM((GCH, 128), jnp.float32),
        pltpu.VMEM((GCH, D), jnp.float32),
        pltpu.VMEM((NCH, D), jnp.float32),
        pltpu.VMEM((NB * 16,), jnp.float32),
        pltpu.VMEM((NNUM, D), jnp.float32),
        pltpu.VMEM((NNUM, D), jnp.float32),
        pltpu.SemaphoreType.DMA,
        pltpu.SemaphoreType.DMA,
    ],
)(_body)


CB = 1024
NBLK = (VG + CB - 1) // CB


def _prepass_body(t_ref, o_ref):
  x = t_ref[...]
  eye = jnp.eye(D, dtype=jnp.float32)
  y = lax.dot_general(x, eye, (((0,), (0,)), ((), ())),
                      preferred_element_type=jnp.float32)
  o_ref[:, 0:D] = y
  o_ref[:, D:128] = jnp.zeros((CB, 128 - D), jnp.float32)


def _prepass(tT):
  return pl.pallas_call(
      _prepass_body,
      grid=(NBLK,),
      in_specs=[pl.BlockSpec((D, CB), lambda i: (0, i))],
      out_specs=pl.BlockSpec((CB, 128), lambda i: (i, 0)),
      out_shape=jax.ShapeDtypeStruct((VG, 128), jnp.float32),
  )(tT)


def kernel(x_categ, x_numer, table, num_weights, num_biases):
  xc2 = x_categ.astype(jnp.int32).reshape(B * F)
  xn = jnp.pad(x_numer.astype(jnp.float32),
               ((0, 0), (0, 16 - NNUM))).reshape(B * 16)
  t128 = _prepass(jnp.swapaxes(table.astype(jnp.float32), 0, 1))
  pat = (jnp.arange(GCH, dtype=jnp.int32) % F) * FIELD_SIZE
  p = jnp.arange(GCH, dtype=jnp.int32)
  pcat = (p // F) * NROWS + (p % F)
  q = jnp.arange(NCH, dtype=jnp.int32)
  pnum = (q // NNUM) * NROWS + F + (q % NNUM)
  out = _embedder(xc2, xn, num_weights.astype(jnp.float32),
                  num_biases.astype(jnp.float32), t128,
                  pat, pcat, pnum)
  return out.reshape(B, NROWS, D)


# speedup vs baseline: 2.0254x; 2.0254x over previous
import functools

import jax
import jax.numpy as jnp
from jax import lax
from jax.experimental import pallas as pl
from jax.experimental.pallas import tpu as pltpu
from jax.experimental.pallas import tpu_sc as plsc

B = 16384
F = 26
NNUM = 13
D = 32
FIELD_SIZE = 100000
NROWS = F + NNUM
NC, NS, L = 2, 16, 16
NW = NC * NS
BPW = B // NW
NB = 16
NCHUNK = BPW // NB
GCH = NB * F
NCH = NB * NNUM
VG = 2600000


def _body(xc_hbm, xn_hbm, w_hbm, bias_hbm, table_hbm, pat_hbm, pcat_hbm,
          pnum_hbm, out_hbm,
          idx_v, pat_v, pcat_v, pnum_v, dcat_v, dnum_v, grp_v, cat_v,
          num_v, xn_v, w_v, b_v, gsem, ssem):
  wid = lax.axis_index("s") * NC + lax.axis_index("c")
  base = wid * BPW

  pltpu.sync_copy(pat_hbm, pat_v)
  pltpu.sync_copy(pcat_hbm, pcat_v)
  pltpu.sync_copy(pnum_hbm, pnum_v)
  pltpu.sync_copy(w_hbm, w_v)
  pltpu.sync_copy(bias_hbm, b_v)

  def chunk(c, carry):
    b0 = base + c * NB
    pltpu.sync_copy(xc_hbm.at[pl.ds(b0 * F, GCH)], idx_v)

    def prep(g, _):
      s = pl.ds(g * L, L)
      idx_v[s] = idx_v[s] + pat_v[s]
      dcat_v[s] = pcat_v[s] + jnp.full((L,), b0 * NROWS, jnp.int32)
      return 0
    lax.fori_loop(0, GCH // L, prep, 0)

    gather = pltpu.async_copy(table_hbm.at[idx_v], grp_v, gsem)

    off = jnp.full((L,), b0 * NROWS, jnp.int32)

    def dnum(g, _):
      s = pl.ds(g * L, L)
      dnum_v[s] = pnum_v[s] + off
      return 0
    lax.fori_loop(0, NCH // L, dnum, 0)

    pltpu.sync_copy(xn_hbm.at[pl.ds(b0 * 16, NB * 16)], xn_v)
    for j in range(NNUM):
      w0 = w_v[j, pl.ds(0, L)]
      w1 = w_v[j, pl.ds(L, L)]
      bb0 = b_v[j, pl.ds(0, L)]
      bb1 = b_v[j, pl.ds(L, L)]

      def nbody(r, _, w0=w0, w1=w1, bb0=bb0, bb1=bb1, j=j):
        xv = xn_v[pl.ds(r * 16, L)]
        xb = xv.at[jnp.full((L,), j, jnp.int32)].get(mode="promise_in_bounds")
        rj = r * NNUM + j
        num_v[rj, pl.ds(0, L)] = xb * w0 + bb0
        num_v[rj, pl.ds(L, L)] = xb * w1 + bb1
        return 0
      lax.fori_loop(0, NB, nbody, 0)

    gather.wait()

    def extract(r, _):
      cat_v[r, pl.ds(0, L)] = grp_v[r, pl.ds(0, L)]
      cat_v[r, pl.ds(L, L)] = grp_v[r, pl.ds(L, L)]
      return 0
    lax.fori_loop(0, GCH, extract, 0)

    sc = pltpu.async_copy(cat_v, out_hbm.at[dcat_v], ssem)
    sn = pltpu.async_copy(num_v, out_hbm.at[dnum_v], ssem)
    sc.wait()
    sn.wait()
    return 0

  lax.fori_loop(0, NCHUNK, chunk, 0)


_embedder = functools.partial(
    pl.kernel,
    out_type=jax.ShapeDtypeStruct((B * NROWS, D), jnp.float32),
    mesh=plsc.VectorSubcoreMesh(
        core_axis_name="c", subcore_axis_name="s",
        num_cores=NC, num_subcores=NS),
    compiler_params=pltpu.CompilerParams(use_tc_tiling_on_sc=False),
    scratch_types=[
        pltpu.VMEM((GCH,), jnp.int32),
        pltpu.VMEM((GCH,), jnp.int32),
        pltpu.VMEM((GCH,), jnp.int32),
        pltpu.VMEM((NCH,), jnp.int32),
        pltpu.VMEM((GCH,), jnp.int32),
        pltpu.VMEM((NCH,), jnp.int32),
        pltpu.VMEM((GCH, 128), jnp.float32),
        pltpu.VMEM((GCH, D), jnp.float32),
        pltpu.VMEM((NCH, D), jnp.float32),
        pltpu.VMEM((NB * 16,), jnp.float32),
        pltpu.VMEM((NNUM, D), jnp.float32),
        pltpu.VMEM((NNUM, D), jnp.float32),
        pltpu.SemaphoreType.DMA,
        pltpu.SemaphoreType.DMA,
    ],
)(_body)


CB = 8192
NBLK = (VG + CB - 1) // CB


def _prepass_body(t_ref, o_ref):
  y = jnp.transpose(t_ref[...], (1, 0))
  o_ref[:, 0:D] = y
  o_ref[:, D:128] = jnp.zeros((CB, 128 - D), jnp.float32)


def _prepass(tT):
  return pl.pallas_call(
      _prepass_body,
      grid=(NBLK,),
      in_specs=[pl.BlockSpec((D, CB), lambda i: (0, i))],
      out_specs=pl.BlockSpec((CB, 128), lambda i: (i, 0)),
      out_shape=jax.ShapeDtypeStruct((VG, 128), jnp.float32),
  )(tT)


def kernel(x_categ, x_numer, table, num_weights, num_biases):
  xc2 = x_categ.astype(jnp.int32).reshape(B * F)
  xn = jnp.pad(x_numer.astype(jnp.float32),
               ((0, 0), (0, 16 - NNUM))).reshape(B * 16)
  t128 = _prepass(jnp.swapaxes(table.astype(jnp.float32), 0, 1))
  pat = (jnp.arange(GCH, dtype=jnp.int32) % F) * FIELD_SIZE
  p = jnp.arange(GCH, dtype=jnp.int32)
  pcat = (p // F) * NROWS + (p % F)
  q = jnp.arange(NCH, dtype=jnp.int32)
  pnum = (q // NNUM) * NROWS + F + (q % NNUM)
  out = _embedder(xc2, xn, num_weights.astype(jnp.float32),
                  num_biases.astype(jnp.float32), t128,
                  pat, pcat, pnum)
  return out.reshape(B, NROWS, D)


# speedup vs baseline: 2.1909x; 1.0817x over previous
import functools

import jax
import jax.numpy as jnp
from jax import lax
from jax.experimental import pallas as pl
from jax.experimental.pallas import tpu as pltpu
from jax.experimental.pallas import tpu_sc as plsc

B = 16384
F = 26
NNUM = 13
D = 32
FIELD_SIZE = 100000
NROWS = F + NNUM
NC, NS, L = 2, 16, 16
NW = NC * NS
BPW = B // NW
NB = 16
NCHUNK = BPW // NB
GCH = NB * F
NCH = NB * NNUM
VG = 2600000


def _body(xc_hbm, xn_hbm, w_hbm, bias_hbm, table_hbm, pat_hbm, pcat_hbm,
          pnum_hbm, out_hbm,
          idx_v, pat_v, pcat_v, pnum_v, dcat_v, dnum_v, grp_v, cat_v,
          num_v, xn_v, w_v, b_v, gsem, ssem):
  wid = lax.axis_index("s") * NC + lax.axis_index("c")
  base = wid * BPW

  pltpu.sync_copy(pat_hbm, pat_v)
  pltpu.sync_copy(pcat_hbm, pcat_v)
  pltpu.sync_copy(pnum_hbm, pnum_v)
  pltpu.sync_copy(w_hbm, w_v)
  pltpu.sync_copy(bias_hbm, b_v)

  def chunk(c, carry):
    b0 = base + c * NB
    pltpu.sync_copy(xc_hbm.at[pl.ds(b0 * F, GCH)], idx_v)

    def prep(g, _):
      s = pl.ds(g * L, L)
      idx_v[s] = idx_v[s] + pat_v[s]
      dcat_v[s] = pcat_v[s] + jnp.full((L,), b0 * NROWS, jnp.int32)
      return 0
    lax.fori_loop(0, GCH // L, prep, 0)

    gather = pltpu.async_copy(table_hbm.at[idx_v], grp_v, gsem)

    off = jnp.full((L,), b0 * NROWS, jnp.int32)

    def dnum(g, _):
      s = pl.ds(g * L, L)
      dnum_v[s] = pnum_v[s] + off
      return 0
    lax.fori_loop(0, NCH // L, dnum, 0)

    pltpu.sync_copy(xn_hbm.at[pl.ds(b0 * 16, NB * 16)], xn_v)
    for j in range(NNUM):
      w0 = w_v[j, pl.ds(0, L)]
      w1 = w_v[j, pl.ds(L, L)]
      bb0 = b_v[j, pl.ds(0, L)]
      bb1 = b_v[j, pl.ds(L, L)]

      def nbody(r, _, w0=w0, w1=w1, bb0=bb0, bb1=bb1, j=j):
        xv = xn_v[pl.ds(r * 16, L)]
        xb = xv.at[jnp.full((L,), j, jnp.int32)].get(mode="promise_in_bounds")
        rj = r * NNUM + j
        num_v[rj, pl.ds(0, L)] = xb * w0 + bb0
        num_v[rj, pl.ds(L, L)] = xb * w1 + bb1
        return 0
      lax.fori_loop(0, NB, nbody, 0)

    gather.wait()

    def extract(r, _):
      cat_v[r, pl.ds(0, L)] = grp_v[r, pl.ds(0, L)]
      cat_v[r, pl.ds(L, L)] = grp_v[r, pl.ds(L, L)]
      return 0
    lax.fori_loop(0, GCH, extract, 0)

    sc = pltpu.async_copy(cat_v, out_hbm.at[dcat_v], ssem)
    sn = pltpu.async_copy(num_v, out_hbm.at[dnum_v], ssem)
    sc.wait()
    sn.wait()
    return 0

  lax.fori_loop(0, NCHUNK, chunk, 0)


_embedder = functools.partial(
    pl.kernel,
    out_type=jax.ShapeDtypeStruct((B * NROWS, D), jnp.float32),
    mesh=plsc.VectorSubcoreMesh(
        core_axis_name="c", subcore_axis_name="s",
        num_cores=NC, num_subcores=NS),
    compiler_params=pltpu.CompilerParams(use_tc_tiling_on_sc=False),
    scratch_types=[
        pltpu.VMEM((GCH,), jnp.int32),
        pltpu.VMEM((GCH,), jnp.int32),
        pltpu.VMEM((GCH,), jnp.int32),
        pltpu.VMEM((NCH,), jnp.int32),
        pltpu.VMEM((GCH,), jnp.int32),
        pltpu.VMEM((NCH,), jnp.int32),
        pltpu.VMEM((GCH, 128), jnp.float32),
        pltpu.VMEM((GCH, D), jnp.float32),
        pltpu.VMEM((NCH, D), jnp.float32),
        pltpu.VMEM((NB * 16,), jnp.float32),
        pltpu.VMEM((NNUM, D), jnp.float32),
        pltpu.VMEM((NNUM, D), jnp.float32),
        pltpu.SemaphoreType.DMA,
        pltpu.SemaphoreType.DMA,
    ],
)(_body)


CB = 20480
NBLK = (VG + CB - 1) // CB


def _prepass_body(t_ref, o_ref):
  y = jnp.transpose(t_ref[...], (1, 0))
  o_ref[:, 0:D] = y
  o_ref[:, D:128] = jnp.zeros((CB, 128 - D), jnp.float32)


def _prepass(tT):
  return pl.pallas_call(
      _prepass_body,
      grid=(NBLK,),
      in_specs=[pl.BlockSpec((D, CB), lambda i: (0, i))],
      out_specs=pl.BlockSpec((CB, 128), lambda i: (i, 0)),
      out_shape=jax.ShapeDtypeStruct((VG, 128), jnp.float32),
  )(tT)


def kernel(x_categ, x_numer, table, num_weights, num_biases):
  xc2 = x_categ.astype(jnp.int32).reshape(B * F)
  xn = jnp.pad(x_numer.astype(jnp.float32),
               ((0, 0), (0, 16 - NNUM))).reshape(B * 16)
  t128 = _prepass(jnp.swapaxes(table.astype(jnp.float32), 0, 1))
  pat = (jnp.arange(GCH, dtype=jnp.int32) % F) * FIELD_SIZE
  p = jnp.arange(GCH, dtype=jnp.int32)
  pcat = (p // F) * NROWS + (p % F)
  q = jnp.arange(NCH, dtype=jnp.int32)
  pnum = (q // NNUM) * NROWS + F + (q % NNUM)
  out = _embedder(xc2, xn, num_weights.astype(jnp.float32),
                  num_biases.astype(jnp.float32), t128,
                  pat, pcat, pnum)
  return out.reshape(B, NROWS, D)


# speedup vs baseline: 2.2065x; 1.0071x over previous
import functools

import jax
import jax.numpy as jnp
from jax import lax
from jax.experimental import pallas as pl
from jax.experimental.pallas import tpu as pltpu
from jax.experimental.pallas import tpu_sc as plsc

B = 16384
F = 26
NNUM = 13
D = 32
FIELD_SIZE = 100000
NROWS = F + NNUM
NC, NS, L = 2, 16, 16
NW = NC * NS
BPW = B // NW
NB = 16
NCHUNK = BPW // NB
GCH = NB * F
NCH = NB * NNUM
VG = 2600000


def _body(xc_hbm, xn_hbm, w_hbm, bias_hbm, table_hbm, pat_hbm, pcat_hbm,
          pnum_hbm, out_hbm,
          idx_v, pat_v, pcat_v, pnum_v, dcat_v, dnum_v, grp_v, cat_v,
          num_v, xn_v, w_v, b_v, gsem, ssem):
  wid = lax.axis_index("s") * NC + lax.axis_index("c")
  base = wid * BPW

  pltpu.sync_copy(pat_hbm, pat_v)
  pltpu.sync_copy(pcat_hbm, pcat_v)
  pltpu.sync_copy(pnum_hbm, pnum_v)
  pltpu.sync_copy(w_hbm, w_v)
  pltpu.sync_copy(bias_hbm, b_v)

  def chunk(c, carry):
    b0 = base + c * NB
    pltpu.sync_copy(xc_hbm.at[pl.ds(b0 * F, GCH)], idx_v)

    def prep(g, _):
      s = pl.ds(g * L, L)
      idx_v[s] = idx_v[s] + pat_v[s]
      dcat_v[s] = pcat_v[s] + jnp.full((L,), b0 * NROWS, jnp.int32)
      return 0
    lax.fori_loop(0, GCH // L, prep, 0)

    gather = pltpu.async_copy(table_hbm.at[idx_v], grp_v, gsem)

    off = jnp.full((L,), b0 * NROWS, jnp.int32)

    def dnum(g, _):
      s = pl.ds(g * L, L)
      dnum_v[s] = pnum_v[s] + off
      return 0
    lax.fori_loop(0, NCH // L, dnum, 0)

    pltpu.sync_copy(xn_hbm.at[pl.ds(b0 * 16, NB * 16)], xn_v)
    for j in range(NNUM):
      w0 = w_v[j, pl.ds(0, L)]
      w1 = w_v[j, pl.ds(L, L)]
      bb0 = b_v[j, pl.ds(0, L)]
      bb1 = b_v[j, pl.ds(L, L)]

      def nbody(r, _, w0=w0, w1=w1, bb0=bb0, bb1=bb1, j=j):
        xv = xn_v[pl.ds(r * 16, L)]
        xb = xv.at[jnp.full((L,), j, jnp.int32)].get(mode="promise_in_bounds")
        rj = r * NNUM + j
        num_v[rj, pl.ds(0, L)] = xb * w0 + bb0
        num_v[rj, pl.ds(L, L)] = xb * w1 + bb1
        return 0
      lax.fori_loop(0, NB, nbody, 0)

    gather.wait()

    def extract(r, _):
      cat_v[r, pl.ds(0, L)] = grp_v[r, pl.ds(0, L)]
      cat_v[r, pl.ds(L, L)] = grp_v[r, pl.ds(L, L)]
      return 0
    lax.fori_loop(0, GCH, extract, 0)

    sc = pltpu.async_copy(cat_v, out_hbm.at[dcat_v], ssem)
    sn = pltpu.async_copy(num_v, out_hbm.at[dnum_v], ssem)
    sc.wait()
    sn.wait()
    return 0

  lax.fori_loop(0, NCHUNK, chunk, 0)


_embedder = functools.partial(
    pl.kernel,
    out_type=jax.ShapeDtypeStruct((B * NROWS, D), jnp.float32),
    mesh=plsc.VectorSubcoreMesh(
        core_axis_name="c", subcore_axis_name="s",
        num_cores=NC, num_subcores=NS),
    compiler_params=pltpu.CompilerParams(use_tc_tiling_on_sc=False),
    scratch_types=[
        pltpu.VMEM((GCH,), jnp.int32),
        pltpu.VMEM((GCH,), jnp.int32),
        pltpu.VMEM((GCH,), jnp.int32),
        pltpu.VMEM((NCH,), jnp.int32),
        pltpu.VMEM((GCH,), jnp.int32),
        pltpu.VMEM((NCH,), jnp.int32),
        pltpu.VMEM((GCH, 128), jnp.float32),
        pltpu.VMEM((GCH, D), jnp.float32),
        pltpu.VMEM((NCH, D), jnp.float32),
        pltpu.VMEM((NB * 16,), jnp.float32),
        pltpu.VMEM((NNUM, D), jnp.float32),
        pltpu.VMEM((NNUM, D), jnp.float32),
        pltpu.SemaphoreType.DMA,
        pltpu.SemaphoreType.DMA,
    ],
)(_body)


CB = 32768
NBLK = (VG + CB - 1) // CB


def _prepass_body(t_ref, o_ref):
  y = jnp.transpose(t_ref[...], (1, 0))
  o_ref[:, 0:D] = y
  o_ref[:, D:128] = jnp.zeros((CB, 128 - D), jnp.float32)


def _prepass(tT):
  return pl.pallas_call(
      _prepass_body,
      grid=(NBLK,),
      in_specs=[pl.BlockSpec((D, CB), lambda i: (0, i))],
      out_specs=pl.BlockSpec((CB, 128), lambda i: (i, 0)),
      out_shape=jax.ShapeDtypeStruct((VG, 128), jnp.float32),
  )(tT)


def kernel(x_categ, x_numer, table, num_weights, num_biases):
  xc2 = x_categ.astype(jnp.int32).reshape(B * F)
  xn = jnp.pad(x_numer.astype(jnp.float32),
               ((0, 0), (0, 16 - NNUM))).reshape(B * 16)
  t128 = _prepass(jnp.swapaxes(table.astype(jnp.float32), 0, 1))
  pat = (jnp.arange(GCH, dtype=jnp.int32) % F) * FIELD_SIZE
  p = jnp.arange(GCH, dtype=jnp.int32)
  pcat = (p // F) * NROWS + (p % F)
  q = jnp.arange(NCH, dtype=jnp.int32)
  pnum = (q // NNUM) * NROWS + F + (q % NNUM)
  out = _embedder(xc2, xn, num_weights.astype(jnp.float32),
                  num_biases.astype(jnp.float32), t128,
                  pat, pcat, pnum)
  return out.reshape(B, NROWS, D)
